# baseline (speedup 1.0000x reference)
import jax
import jax.numpy as jnp
from jax import lax
from jax.experimental import pallas as pl
from jax.experimental.pallas import tpu as pltpu
from jax.experimental.pallas import tpu_sc as plsc

N_NODES = 10000
D = 128
DH = D // 2
E_EDGES = 320000

NUM_CORES = 2
NUM_SUBCORES = 16
EDGES_PER_T = E_EDGES // NUM_SUBCORES
CHUNK = 80
CHUNKS_PER_T = EDGES_PER_T // CHUNK
ACC_ROWS = 10240
ROWS_PER_TILE = ACC_ROWS // NUM_SUBCORES

NBUF = 4
ROUNDS = (CHUNKS_PER_T - 1) // NBUF
TAIL = CHUNKS_PER_T - ROUNDS * NBUF


def _sc_body(xs_hbm, src_hbm, dst_hbm, zero_hbm, part_hbm,
             src_v, dst_v, b0, b1, b2, b3, acc_sh,
             sem_i, g0, g1, g2, g3, s0, s1, s2, s3):
    bufs = (b0, b1, b2, b3)
    gsems = (g0, g1, g2, g3)
    ssems = (s0, s1, s2, s3)
    c = lax.axis_index("c")
    s = lax.axis_index("s")
    x_half = xs_hbm.at[c]

    cp_src = pltpu.async_copy(src_hbm.at[s], src_v, sem_i)
    cp_dst = pltpu.async_copy(dst_hbm.at[s], dst_v, sem_i)
    row0 = s * ROWS_PER_TILE
    pltpu.sync_copy(zero_hbm.at[pl.ds(row0, ROWS_PER_TILE)],
                    acc_sh.at[pl.ds(row0, ROWS_PER_TILE)])
    cp_src.wait()
    cp_dst.wait()
    plsc.subcore_barrier()

    def gather(i, j):
        pltpu.async_copy(x_half.at[src_v.at[pl.ds(i * CHUNK, CHUNK)]],
                         bufs[j], gsems[j])

    def drain_gather(j):
        pltpu.make_async_copy(x_half.at[pl.ds(0, CHUNK)], bufs[j], gsems[j]).wait()

    def scatter(i, j):
        pltpu.async_copy(bufs[j], acc_sh.at[dst_v.at[i]], ssems[j], add=True)

    def drain_scatter(j):
        pltpu.make_async_copy(bufs[j], acc_sh.at[pl.ds(0, CHUNK)], ssems[j]).wait()

    for j in range(NBUF):
        gather(j, j)

    def round_fn(k, carry):
        i0 = k * NBUF
        for j in range(NBUF):
            drain_gather(j)
            scatter(i0 + j, j)
        for j in range(NBUF):
            drain_scatter(j)
            nxt = i0 + NBUF + j

            @pl.when(nxt < CHUNKS_PER_T)
            def _():
                gather(nxt, j)
        return carry

    lax.fori_loop(0, ROUNDS, round_fn, 0)
    for j in range(TAIL):
        drain_gather(j)
        scatter(ROUNDS * NBUF + j, j)
    for j in range(TAIL):
        drain_scatter(j)
    plsc.subcore_barrier()

    pltpu.sync_copy(acc_sh.at[pl.ds(row0, ROWS_PER_TILE)],
                    part_hbm.at[c, pl.ds(row0, ROWS_PER_TILE)])


@jax.jit
def _sc_aggregate(xs, src_r, dst_r, zeros):
    mesh = plsc.VectorSubcoreMesh(core_axis_name="c", subcore_axis_name="s")
    return pl.kernel(
        _sc_body,
        out_type=jax.ShapeDtypeStruct((NUM_CORES, ACC_ROWS, DH), jnp.float32),
        mesh=mesh,
        compiler_params=pltpu.CompilerParams(use_tc_tiling_on_sc=False),
        scratch_types=[
            pltpu.VMEM((EDGES_PER_T,), jnp.int32),
            pltpu.VMEM((CHUNKS_PER_T, CHUNK), jnp.int32),
            pltpu.VMEM((CHUNK, DH), jnp.float32),
            pltpu.VMEM((CHUNK, DH), jnp.float32),
            pltpu.VMEM((CHUNK, DH), jnp.float32),
            pltpu.VMEM((CHUNK, DH), jnp.float32),
            pltpu.VMEM_SHARED((ACC_ROWS, DH), jnp.float32),
            pltpu.SemaphoreType.DMA,
            pltpu.SemaphoreType.DMA,
            pltpu.SemaphoreType.DMA,
            pltpu.SemaphoreType.DMA,
            pltpu.SemaphoreType.DMA,
            pltpu.SemaphoreType.DMA,
            pltpu.SemaphoreType.DMA,
            pltpu.SemaphoreType.DMA,
            pltpu.SemaphoreType.DMA,
        ],
    )(xs, src_r, dst_r, zeros)


ROW_BLK = 2000


def _tc_body(p_ref, x_ref, wrel_ref, wroot_ref, b_ref, o_ref):
    o_ref[...] = (
        jnp.dot(p_ref[0], wrel_ref[0:DH, :], preferred_element_type=jnp.float32)
        + jnp.dot(p_ref[1], wrel_ref[DH:D, :], preferred_element_type=jnp.float32)
        + jnp.dot(x_ref[...], wroot_ref[...], preferred_element_type=jnp.float32)
        + b_ref[...]
    )


@jax.jit
def _tc_combine(parts, x, wrel_t, wroot_t, b2):
    grid = N_NODES // ROW_BLK
    return pl.pallas_call(
        _tc_body,
        grid=(grid,),
        in_specs=[
            pl.BlockSpec((NUM_CORES, ROW_BLK, DH), lambda i: (0, i, 0)),
            pl.BlockSpec((ROW_BLK, D), lambda i: (i, 0)),
            pl.BlockSpec((D, D), lambda i: (0, 0)),
            pl.BlockSpec((D, D), lambda i: (0, 0)),
            pl.BlockSpec((1, D), lambda i: (0, 0)),
        ],
        out_specs=pl.BlockSpec((ROW_BLK, D), lambda i: (i, 0)),
        out_shape=jax.ShapeDtypeStruct((N_NODES, D), jnp.float32),
    )(parts, x, wrel_t, wroot_t, b2)


def kernel(x, edge_index, W_rel, W_root, b):
    src = edge_index[0].reshape(NUM_SUBCORES, EDGES_PER_T)
    dst = edge_index[1].reshape(NUM_SUBCORES, CHUNKS_PER_T, CHUNK)
    xs = jnp.moveaxis(x.reshape(N_NODES, NUM_CORES, DH), 1, 0)
    zeros = jnp.zeros((ACC_ROWS, DH), dtype=jnp.float32)
    parts = _sc_aggregate(xs, src, dst, zeros)
    return _tc_combine(parts, x, W_rel.T, W_root.T, b.reshape(1, D))
